# Initial kernel scaffold; baseline (speedup 1.0000x reference)
import jax
import jax.numpy as jnp
from jax.experimental import pallas as pl


def kernel(x, gemm_edges, W0, W1, gamma1, beta1):
    raise NotImplementedError("write your pallas kernel here")


# speedup vs baseline: 3.6675x; 3.6675x over previous
import functools

import jax
import jax.numpy as jnp
from jax import lax
from jax.experimental import pallas as pl
from jax.experimental.pallas import tpu as pltpu
from jax.experimental.pallas import tpu_sc as plsc

_NEG = 0.01
_EPS = 1e-5
_NC = 2
_NW = 32
_CH = 80
_BLK = 640


def _sc_gather_call(table, idx):
    M = idx.shape[0]
    D = table.shape[1]
    per_w = M // _NW
    n_ch = per_w // _CH
    mesh = plsc.VectorSubcoreMesh(core_axis_name="c", subcore_axis_name="s")

    @functools.partial(
        pl.kernel,
        mesh=mesh,
        out_type=jax.ShapeDtypeStruct((M, D), jnp.float32),
        scratch_types=[
            pltpu.VMEM((_CH,), jnp.int32),
            pltpu.VMEM((_CH, D), jnp.float32),
            pltpu.SemaphoreType.DMA,
        ],
    )
    def k(table_hbm, idx_hbm, out_hbm, idx_v, rows_v, sem):
        wid = lax.axis_index("s") * _NC + lax.axis_index("c")
        base = wid * per_w

        def body(i, c):
            off = base + i * _CH
            pltpu.sync_copy(idx_hbm.at[pl.ds(off, _CH)], idx_v)
            pltpu.async_copy(table_hbm.at[idx_v], rows_v, sem).wait()
            pltpu.sync_copy(rows_v, out_hbm.at[pl.ds(off, _CH)])
            return c

        lax.fori_loop(0, n_ch, body, 0)

    return k(table, idx)


def _leaky(t):
    return jnp.where(t >= 0, t, _NEG * t)


def _combine(f0, g1, g2, g3, g4, w_ref):
    s13 = g1 + g3
    s24 = g2 + g4
    d13 = jnp.abs(g1 - g3)
    d24 = jnp.abs(g2 - g4)
    x5 = s13 + s24
    x6 = 0.5 * (s13 * s13 + s24 * s24 + d13 * d13 + d24 * d24) - 0.25 * (x5 * x5)

    def dot(a, k):
        return jnp.dot(a, w_ref[k], preferred_element_type=jnp.float32)

    return (dot(f0, 0) + dot(s13, 1) + dot(s24, 2)
            + dot(d13, 3) + dot(d24, 4) + dot(x6, 5))


def _conv0_body(x_ref, g_ref, w_ref, h_ref, st_ref):
    h = _combine(x_ref[...], g_ref[0], g_ref[1], g_ref[2], g_ref[3], w_ref)
    h_ref[...] = h
    y = _leaky(h)

    @pl.when(pl.program_id(0) == 0)
    def _():
        st_ref[...] = jnp.zeros_like(st_ref)

    st_ref[0:1, :] += jnp.sum(y, axis=0, keepdims=True)
    st_ref[1:2, :] += jnp.sum(y * y, axis=0, keepdims=True)


def _conv1_body(h0_ref, g_ref, w_ref, ab_ref, o_ref):
    a = ab_ref[0:1, :]
    b = ab_ref[1:2, :]

    def norm(t):
        return _leaky(t) * a + b

    h0 = h0_ref[...]
    h2 = _combine(norm(h0), norm(g_ref[0]), norm(g_ref[1]),
                  norm(g_ref[2]), norm(g_ref[3]), w_ref)
    r = h2 + h0
    o_ref[...] = _leaky(r)


def _tc_conv0(xt, g, wc, interpret=False):
    E, C = xt.shape
    nb = E // _BLK
    return pl.pallas_call(
        _conv0_body,
        grid=(nb,),
        in_specs=[
            pl.BlockSpec((_BLK, C), lambda i: (i, 0)),
            pl.BlockSpec((4, _BLK, C), lambda i: (0, i, 0)),
            pl.BlockSpec((6, C, C), lambda i: (0, 0, 0)),
        ],
        out_specs=[
            pl.BlockSpec((_BLK, C), lambda i: (i, 0)),
            pl.BlockSpec((8, C), lambda i: (0, 0)),
        ],
        out_shape=[
            jax.ShapeDtypeStruct((E, C), jnp.float32),
            jax.ShapeDtypeStruct((8, C), jnp.float32),
        ],
        compiler_params=pltpu.CompilerParams(
            dimension_semantics=("arbitrary",)),
        interpret=interpret,
    )(xt, g, wc)


def _tc_conv1(h0, g, wc, ab, interpret=False):
    E, C = h0.shape
    nb = E // _BLK
    return pl.pallas_call(
        _conv1_body,
        grid=(nb,),
        in_specs=[
            pl.BlockSpec((_BLK, C), lambda i: (i, 0)),
            pl.BlockSpec((4, _BLK, C), lambda i: (0, i, 0)),
            pl.BlockSpec((6, C, C), lambda i: (0, 0, 0)),
            pl.BlockSpec((8, C), lambda i: (0, 0)),
        ],
        out_specs=pl.BlockSpec((_BLK, C), lambda i: (i, 0)),
        out_shape=jax.ShapeDtypeStruct((E, C), jnp.float32),
        compiler_params=pltpu.CompilerParams(
            dimension_semantics=("arbitrary",)),
        interpret=interpret,
    )(h0, g, wc, ab)


def _prep_w(W):
    Ws = W[:, :, 0, :]
    taps = [Ws[:, :, 0],
            Ws[:, :, 1] + Ws[:, :, 5],
            Ws[:, :, 2] + Ws[:, :, 5],
            Ws[:, :, 3],
            Ws[:, :, 4],
            Ws[:, :, 6]]
    return jnp.stack([t.T for t in taps])


def kernel(x, gemm_edges, W0, W1, gamma1, beta1):
    xs = x[0, :, :, 0]
    C, E = xs.shape
    xt = xs.T
    idx = gemm_edges[0].T.reshape(-1)
    wc0 = _prep_w(W0)
    wc1 = _prep_w(W1)

    g0 = _sc_gather_call(xt, idx).reshape(4, E, C)
    h0, stats = _tc_conv0(xt, g0, wc0)

    mean = stats[0] / E
    var = stats[1] / E - mean * mean
    a = gamma1 * lax.rsqrt(var + _EPS)
    b = beta1 - mean * a
    ab = jnp.zeros((8, C), jnp.float32).at[0].set(a).at[1].set(b)

    g1 = _sc_gather_call(h0, idx).reshape(4, E, C)
    outT = _tc_conv1(h0, g1, wc1, ab)
    return outT.T[None, :, :, None]


# speedup vs baseline: 5.1464x; 1.4032x over previous
import functools

import jax
import jax.numpy as jnp
from jax import lax
from jax.experimental import pallas as pl
from jax.experimental.pallas import tpu as pltpu
from jax.experimental.pallas import tpu_sc as plsc

_NEG = 0.01
_EPS = 1e-5
_NC = 2
_NW = 32
_CH = 80
_BLK = 640


def _sc_gather_call(table, idx):
    M = idx.shape[0] - 2 * _CH
    D = table.shape[1]
    dt = table.dtype
    per_w = M // _NW
    n_ch = per_w // _CH
    n_pair = n_ch // 2
    mesh = plsc.VectorSubcoreMesh(core_axis_name="c", subcore_axis_name="s")

    @functools.partial(
        pl.kernel,
        mesh=mesh,
        out_type=jax.ShapeDtypeStruct((M, D), dt),
        scratch_types=[
            pltpu.VMEM((_CH,), jnp.int32),
            pltpu.VMEM((_CH,), jnp.int32),
            pltpu.VMEM((_CH, D), dt),
            pltpu.VMEM((_CH, D), dt),
            pltpu.SemaphoreType.DMA,
            pltpu.SemaphoreType.DMA,
            pltpu.SemaphoreType.DMA,
            pltpu.SemaphoreType.DMA,
            pltpu.SemaphoreType.DMA,
            pltpu.SemaphoreType.DMA,
        ],
    )
    def k(table_hbm, idx_hbm, out_hbm,
          idx0, idx1, rows0, rows1, si0, si1, sg0, sg1, ss0, ss1):
        wid = lax.axis_index("s") * _NC + lax.axis_index("c")
        base = wid * per_w

        def ld_idx(i, buf, sem):
            pltpu.async_copy(idx_hbm.at[pl.ds(base + i * _CH, _CH)], buf, sem)

        def gather(buf_idx, buf_rows, sem):
            pltpu.async_copy(table_hbm.at[buf_idx], buf_rows, sem)

        def store(i, buf_rows, sem):
            pltpu.async_copy(buf_rows, out_hbm.at[pl.ds(base + i * _CH, _CH)], sem)

        def w_idx(buf, sem):
            pltpu.make_async_copy(idx_hbm.at[pl.ds(0, _CH)], buf, sem).wait()

        def w_gat(buf_idx, buf_rows, sem):
            pltpu.make_async_copy(table_hbm.at[buf_idx], buf_rows, sem).wait()

        def w_st(buf_rows, sem):
            pltpu.make_async_copy(buf_rows, out_hbm.at[pl.ds(0, _CH)], sem).wait()

        ld_idx(0, idx0, si0)
        ld_idx(1, idx1, si1)
        w_idx(idx0, si0)
        gather(idx0, rows0, sg0)
        w_idx(idx1, si1)
        gather(idx1, rows1, sg1)
        w_gat(idx0, rows0, sg0)
        store(0, rows0, ss0)
        ld_idx(2, idx0, si0)
        w_gat(idx1, rows1, sg1)
        store(1, rows1, ss1)
        ld_idx(3, idx1, si1)

        def body(j, c):
            i0 = 2 * j
            w_idx(idx0, si0)
            w_st(rows0, ss0)
            gather(idx0, rows0, sg0)
            w_idx(idx1, si1)
            w_st(rows1, ss1)
            gather(idx1, rows1, sg1)
            w_gat(idx0, rows0, sg0)
            store(i0, rows0, ss0)
            ld_idx(i0 + 2, idx0, si0)
            w_gat(idx1, rows1, sg1)
            store(i0 + 1, rows1, ss1)
            ld_idx(i0 + 3, idx1, si1)
            return c

        lax.fori_loop(1, n_pair, body, 0)
        w_idx(idx0, si0)
        w_idx(idx1, si1)
        w_st(rows0, ss0)
        w_st(rows1, ss1)

    return k(table, idx)


def _leaky(t):
    return jnp.where(t >= 0, t, _NEG * t)


def _combine(f0, g1, g2, g3, g4, w_ref):
    s13 = g1 + g3
    s24 = g2 + g4
    d13 = jnp.abs(g1 - g3)
    d24 = jnp.abs(g2 - g4)
    x5 = s13 + s24
    x6 = 0.5 * (s13 * s13 + s24 * s24 + d13 * d13 + d24 * d24) - 0.25 * (x5 * x5)

    def dot(a, k):
        return jnp.dot(a, w_ref[k], preferred_element_type=jnp.float32)

    return (dot(f0, 0) + dot(s13, 1) + dot(s24, 2)
            + dot(d13, 3) + dot(d24, 4) + dot(x6, 5))


def _conv0_body(x_ref, g_ref, w_ref, h_ref, st_ref):
    h = _combine(x_ref[...], g_ref[0], g_ref[1], g_ref[2], g_ref[3], w_ref)
    h_ref[...] = h
    y = _leaky(h)

    @pl.when(pl.program_id(0) == 0)
    def _():
        st_ref[...] = jnp.zeros_like(st_ref)

    st_ref[0:1, :] += jnp.sum(y, axis=0, keepdims=True)
    st_ref[1:2, :] += jnp.sum(y * y, axis=0, keepdims=True)


def _conv1_body(h0_ref, g_ref, w_ref, ab_ref, o_ref):
    a = ab_ref[0:1, :]
    b = ab_ref[1:2, :]

    def norm(t):
        return _leaky(t) * a + b

    h0 = h0_ref[...]
    h2 = _combine(norm(h0), norm(g_ref[0]), norm(g_ref[1]),
                  norm(g_ref[2]), norm(g_ref[3]), w_ref)
    r = h2 + h0
    o_ref[...] = _leaky(r)


def _tc_conv0(xt, g, wc, interpret=False):
    E, C = xt.shape
    nb = E // _BLK
    return pl.pallas_call(
        _conv0_body,
        grid=(nb,),
        in_specs=[
            pl.BlockSpec((_BLK, C), lambda i: (i, 0)),
            pl.BlockSpec((4, _BLK, C), lambda i: (0, i, 0)),
            pl.BlockSpec((6, C, C), lambda i: (0, 0, 0)),
        ],
        out_specs=[
            pl.BlockSpec((_BLK, C), lambda i: (i, 0)),
            pl.BlockSpec((8, C), lambda i: (0, 0)),
        ],
        out_shape=[
            jax.ShapeDtypeStruct((E, C), jnp.float32),
            jax.ShapeDtypeStruct((8, C), jnp.float32),
        ],
        compiler_params=pltpu.CompilerParams(
            dimension_semantics=("arbitrary",)),
        interpret=interpret,
    )(xt, g, wc)


def _tc_conv1(h0, g, wc, ab, interpret=False):
    E, C = h0.shape
    nb = E // _BLK
    return pl.pallas_call(
        _conv1_body,
        grid=(nb,),
        in_specs=[
            pl.BlockSpec((_BLK, C), lambda i: (i, 0)),
            pl.BlockSpec((4, _BLK, C), lambda i: (0, i, 0)),
            pl.BlockSpec((6, C, C), lambda i: (0, 0, 0)),
            pl.BlockSpec((8, C), lambda i: (0, 0)),
        ],
        out_specs=pl.BlockSpec((_BLK, C), lambda i: (i, 0)),
        out_shape=jax.ShapeDtypeStruct((E, C), jnp.float32),
        compiler_params=pltpu.CompilerParams(
            dimension_semantics=("arbitrary",)),
        interpret=interpret,
    )(h0, g, wc, ab)


def _prep_w(W):
    Ws = W[:, :, 0, :]
    taps = [Ws[:, :, 0],
            Ws[:, :, 1] + Ws[:, :, 5],
            Ws[:, :, 2] + Ws[:, :, 5],
            Ws[:, :, 3],
            Ws[:, :, 4],
            Ws[:, :, 6]]
    return jnp.stack([t.T for t in taps])


def kernel(x, gemm_edges, W0, W1, gamma1, beta1):
    xs = x[0, :, :, 0]
    C, E = xs.shape
    xt = xs.T
    idx = gemm_edges[0].T.reshape(-1)
    idx = jnp.concatenate([idx, jnp.zeros((2 * _CH,), jnp.int32)])
    wc0 = _prep_w(W0)
    wc1 = _prep_w(W1)

    g0 = _sc_gather_call(xt, idx).reshape(4, E, C)
    h0, stats = _tc_conv0(xt, g0, wc0)

    mean = stats[0] / E
    var = stats[1] / E - mean * mean
    a = gamma1 * lax.rsqrt(var + _EPS)
    b = beta1 - mean * a
    ab = jnp.zeros((8, C), jnp.float32).at[0].set(a).at[1].set(b)

    g1 = _sc_gather_call(h0, idx).reshape(4, E, C)
    outT = _tc_conv1(h0, g1, wc1, ab)
    return outT.T[None, :, :, None]
